# baseline (speedup 1.0000x reference)
import functools
import math

import jax
import jax.numpy as jnp
from jax import lax
from jax.experimental import pallas as pl
from jax.experimental.pallas import tpu as pltpu
from jax.experimental.pallas import tpu_sc as plsc

B = 16
G = 64
P = 500
PP = 512
L = 16
GS = 4 * G + 4 * L

HALF_LN_ALPHA = 0.5 * math.log(1.0 - 0.01)

_mesh = plsc.VectorSubcoreMesh(core_axis_name="c", subcore_axis_name="s")


@functools.partial(
    pl.kernel,
    mesh=_mesh,
    compiler_params=pltpu.CompilerParams(needs_layout_passes=False),
    out_type=jax.ShapeDtypeStruct((B * 4 * PP,), jnp.float32),
    scratch_types=[
        pltpu.VMEM((GS,), jnp.float32),
        pltpu.VMEM((2 * PP,), jnp.int32),
        pltpu.VMEM((2 * PP,), jnp.float32),
        pltpu.VMEM((2 * PP,), jnp.float32),
        pltpu.SemaphoreType.DMA,
        pltpu.SemaphoreType.DMA,
        pltpu.SemaphoreType.DMA,
    ],
)
def _noised_gt_sc(gs_hbm, it_hbm, nz_hbm, out_hbm,
                  gs_v, it_v, nz_v, o_v, s0, s1, s2):
    wid = lax.axis_index("s") * 2 + lax.axis_index("c")
    b = wid // 2
    ph = wid % 2
    cps = [
        pltpu.async_copy(gs_hbm.at[pl.ds(b * GS, GS)], gs_v, s0),
        pltpu.async_copy(it_hbm.at[pl.ds(b * 2 * PP, 2 * PP)], it_v, s1),
        pltpu.async_copy(
            nz_hbm.at[pl.ds(b * 4 * PP + ph * 2 * PP, 2 * PP)], nz_v, s2),
    ]
    for cp in cps:
        cp.wait()

    @pl.loop(0, PP // (2 * L))
    def _j(j):
        li = it_v[pl.ds(ph * (PP // 2) + j * L, L)]
        tf = it_v[pl.ds(PP + ph * (PP // 2) + j * L, L)].astype(jnp.float32)
        sqrt_a = jnp.exp(tf * HALF_LN_ALPHA)
        x = 1.0 - sqrt_a * sqrt_a
        y = jnp.where(x > 0.215, 0.681, jnp.where(x > 0.0464, 0.316, 0.1465))
        y = 0.5 * (y + x / y)
        y = 0.5 * (y + x / y)
        y = 0.5 * (y + x / y)
        sqrt_b = jnp.where(x > 0.0, y, 0.0)
        po = (j // 8) * (4 * 128) + (j % 8) * L
        for c in range(4):
            box = plsc.load_gather(gs_v, [li + c * G])
            s = gs_v[pl.ds(4 * G + c * L, L)]
            nzc = nz_v[pl.ds(po + c * 128, L)]
            o_v[pl.ds(po + c * 128, L)] = s * (box * sqrt_a + nzc * sqrt_b)

    pltpu.sync_copy(o_v, out_hbm.at[pl.ds(b * 4 * PP + ph * 2 * PP, 2 * PP)])


def kernel(gt_boxes, scales, sampled_indices, t, noise):
    gs = jnp.concatenate(
        [gt_boxes.transpose(0, 2, 1).reshape(B, 4 * G),
         jnp.broadcast_to(scales[:, :, None], (B, 4, L)).reshape(B, 4 * L)],
        axis=1).reshape(-1)
    it = jnp.concatenate(
        [jnp.pad(sampled_indices.astype(jnp.int32), ((0, 0), (0, PP - P))),
         jnp.pad(t.astype(jnp.int32), ((0, 0), (0, PP - P)))],
        axis=1).reshape(-1)
    nz4 = (jnp.pad(noise, ((0, 0), (0, PP - P), (0, 0)))
           .reshape(B, 4, 128, 4).transpose(0, 1, 3, 2).reshape(-1))
    out = _noised_gt_sc(gs, it, nz4)
    o = out.reshape(B, 4, 4, 128).transpose(0, 2, 1, 3).reshape(B, 4, PP)
    return o[:, :, :P].transpose(0, 2, 1)


# speedup vs baseline: 1.0128x; 1.0128x over previous
import functools
import math

import jax
import jax.numpy as jnp
from jax import lax
from jax.experimental import pallas as pl
from jax.experimental.pallas import tpu as pltpu
from jax.experimental.pallas import tpu_sc as plsc

B = 16
G = 64
P = 500
PP = 512
L = 16
GS = 4 * G + 4 * L

HALF_LN_ALPHA = 0.5 * math.log(1.0 - 0.01)

_mesh = plsc.VectorSubcoreMesh(
    core_axis_name="c", subcore_axis_name="s", num_cores=1)


@functools.partial(
    pl.kernel,
    mesh=_mesh,
    compiler_params=pltpu.CompilerParams(needs_layout_passes=False),
    out_type=jax.ShapeDtypeStruct((B * 4 * PP,), jnp.float32),
    scratch_types=[
        pltpu.VMEM((GS,), jnp.float32),
        pltpu.VMEM((2 * PP,), jnp.int32),
        pltpu.VMEM((4 * PP,), jnp.float32),
        pltpu.VMEM((4 * PP,), jnp.float32),
        pltpu.SemaphoreType.DMA,
        pltpu.SemaphoreType.DMA,
        pltpu.SemaphoreType.DMA,
    ],
)
def _noised_gt_sc(gs_hbm, it_hbm, nz_hbm, out_hbm,
                  gs_v, it_v, nz_v, o_v, s0, s1, s2):
    b = lax.axis_index("s")
    cps = [
        pltpu.async_copy(gs_hbm.at[pl.ds(b * GS, GS)], gs_v, s0),
        pltpu.async_copy(it_hbm.at[pl.ds(b * 2 * PP, 2 * PP)], it_v, s1),
        pltpu.async_copy(nz_hbm.at[pl.ds(b * 4 * PP, 4 * PP)], nz_v, s2),
    ]
    for cp in cps:
        cp.wait()

    @pl.loop(0, PP // L)
    def _j(j):
        li = it_v[pl.ds(j * L, L)]
        tf = it_v[pl.ds(PP + j * L, L)].astype(jnp.float32)
        sqrt_a = jnp.exp(tf * HALF_LN_ALPHA)
        x = 1.0 - sqrt_a * sqrt_a
        y = jnp.where(x > 0.215, 0.681, jnp.where(x > 0.0464, 0.316, 0.1465))
        y = 0.5 * (y + x / y)
        y = 0.5 * (y + x / y)
        y = 0.5 * (y + x / y)
        sqrt_b = jnp.where(x > 0.0, y, 0.0)
        po = (j // 8) * (4 * 128) + (j % 8) * L
        for c in range(4):
            box = plsc.load_gather(gs_v, [li + c * G])
            s = gs_v[pl.ds(4 * G + c * L, L)]
            nzc = nz_v[pl.ds(po + c * 128, L)]
            o_v[pl.ds(po + c * 128, L)] = s * (box * sqrt_a + nzc * sqrt_b)

    pltpu.sync_copy(o_v, out_hbm.at[pl.ds(b * 4 * PP, 4 * PP)])


def kernel(gt_boxes, scales, sampled_indices, t, noise):
    gs = jnp.concatenate(
        [gt_boxes.transpose(0, 2, 1).reshape(B, 4 * G),
         jnp.broadcast_to(scales[:, :, None], (B, 4, L)).reshape(B, 4 * L)],
        axis=1).reshape(-1)
    it = jnp.concatenate(
        [jnp.pad(sampled_indices.astype(jnp.int32), ((0, 0), (0, PP - P))),
         jnp.pad(t.astype(jnp.int32), ((0, 0), (0, PP - P)))],
        axis=1).reshape(-1)
    nz4 = (jnp.pad(noise, ((0, 0), (0, PP - P), (0, 0)))
           .reshape(B, 4, 128, 4).transpose(0, 1, 3, 2).reshape(-1))
    out = _noised_gt_sc(gs, it, nz4)
    o = out.reshape(B, 4, 4, 128).transpose(0, 2, 1, 3).reshape(B, 4, PP)
    return o[:, :, :P].transpose(0, 2, 1)
